# Initial kernel scaffold; baseline (speedup 1.0000x reference)
import jax
import jax.numpy as jnp
from jax.experimental import pallas as pl


def kernel(x, adj_t, W1l, W1r, b1, W2l, W2r, b2, W3l, W3r, b3):
    raise NotImplementedError("write your pallas kernel here")


# speedup vs baseline: 2.2914x; 2.2914x over previous
import jax
import jax.numpy as jnp
from jax import lax
from jax.experimental import pallas as pl
from jax.experimental.pallas import tpu as pltpu
from jax.experimental.pallas import tpu_sc as plsc

N = 10000
E = 320000
F = 128
H = 128
C = 47
CP = 64

NC = 2
NS = 16
NW = NC * NS
B = 128
CPW = 80
NCHUNK = NW * CPW
EP = NCHUNK * B
NPAD = 10240
RPT = NPAD // NS
DW = 128
ZR = 64

BN = 1000


_sc_cache = {}


def _mesh():
    return plsc.VectorSubcoreMesh(core_axis_name="c", subcore_axis_name="s",
                                  num_cores=NC, num_subcores=NS)


def _zero_slab(zb, width):
    zero16 = jnp.zeros((16,), jnp.float32)

    @pl.loop(0, zb.shape[0])
    def _(r):
        @pl.loop(0, width, step=16)
        def _(cc):
            zb[r, pl.ds(cc, 16)] = zero16


def _zero_acc(zb, acc, sid):
    @pl.loop(0, RPT, step=ZR)
    def _(r0):
        pltpu.sync_copy(zb, acc.at[pl.ds(sid * RPT + r0, ZR)])


def _sc_agg_builder(width):
    if ("agg", width) in _sc_cache:
        return _sc_cache[("agg", width)]

    scratch = [
        pltpu.VMEM((2, B), jnp.int32),
        pltpu.VMEM((B, width), jnp.float32),
        pltpu.VMEM((ZR, width), jnp.float32),
        pltpu.VMEM_SHARED((NPAD, width), jnp.float32),
        pltpu.SemaphoreType.DMA,
    ]

    def body(y_hbm, eidx_hbm, agg_out, idxb, rowsb, zb, acc, sem):
        cid = lax.axis_index("c")
        sid = lax.axis_index("s")
        wid = sid * NC + cid

        _zero_slab(zb, width)
        _zero_acc(zb, acc, sid)
        plsc.subcore_barrier()

        base = wid * CPW

        @pl.loop(0, CPW)
        def _(i):
            pltpu.sync_copy(eidx_hbm.at[base + i], idxb)
            pltpu.async_copy(y_hbm.at[idxb.at[0]], rowsb, sem).wait()
            pltpu.sync_copy(rowsb, acc.at[idxb.at[1]], add=True)

        plsc.subcore_barrier()

        @pl.loop(0, RPT, step=ZR)
        def _(r0):
            row = sid * RPT + r0
            pltpu.sync_copy(acc.at[pl.ds(row, ZR)],
                            agg_out.at[cid, pl.ds(row, ZR)])

    fn = pl.kernel(body,
                   out_type=jax.ShapeDtypeStruct((NC, NPAD, width),
                                                 jnp.float32),
                   mesh=_mesh(), scratch_types=scratch)
    _sc_cache[("agg", width)] = fn
    return fn


def _sc_deg_builder():
    if "deg" in _sc_cache:
        return _sc_cache["deg"]

    scratch = [
        pltpu.VMEM((2, B), jnp.int32),
        pltpu.VMEM((B, H), jnp.float32),
        pltpu.VMEM((ZR, H), jnp.float32),
        pltpu.VMEM_SHARED((NPAD, H), jnp.float32),
    ]

    def body(eidx_hbm, deg_out, idxb, onesb, zb, acc):
        cid = lax.axis_index("c")
        sid = lax.axis_index("s")
        wid = sid * NC + cid

        _zero_slab(zb, H)
        one16 = jnp.ones((16,), jnp.float32)

        @pl.loop(0, B)
        def _(r):
            @pl.loop(0, H, step=16)
            def _(cc):
                onesb[r, pl.ds(cc, 16)] = one16

        _zero_acc(zb, acc, sid)
        plsc.subcore_barrier()

        base = wid * CPW

        @pl.loop(0, CPW)
        def _(i):
            pltpu.sync_copy(eidx_hbm.at[base + i], idxb)
            pltpu.sync_copy(onesb, acc.at[idxb.at[1]], add=True)

        plsc.subcore_barrier()

        @pl.loop(0, RPT, step=ZR)
        def _(r0):
            row = sid * RPT + r0
            pltpu.sync_copy(acc.at[pl.ds(row, ZR), pl.ds(0, DW)],
                            deg_out.at[cid, pl.ds(row, ZR)])

    fn = pl.kernel(body,
                   out_type=jax.ShapeDtypeStruct((NC, NPAD, DW),
                                                 jnp.float32),
                   mesh=_mesh(), scratch_types=scratch)
    _sc_cache["deg"] = fn
    return fn


def _sc_deg(eidx):
    return _sc_deg_builder()(eidx)


def _sc_agg(y, eidx):
    return _sc_agg_builder(H)(y, eidx)


def _tc_in_body(x_ref, w_ref, yl_ref, yr_ref):
    y = jnp.dot(x_ref[...], w_ref[...], preferred_element_type=jnp.float32)
    yl_ref[...] = y[:, :H]
    yr_ref[...] = y[:, H:]


def _tc_in(x, wcat):
    return pl.pallas_call(
        _tc_in_body,
        grid=(N // BN,),
        in_specs=[pl.BlockSpec((BN, F), lambda i: (i, 0)),
                  pl.BlockSpec((F, 2 * H), lambda i: (0, 0))],
        out_specs=[pl.BlockSpec((BN, H), lambda i: (i, 0)),
                   pl.BlockSpec((BN, H), lambda i: (i, 0))],
        out_shape=[jax.ShapeDtypeStruct((N, H), jnp.float32),
                   jax.ShapeDtypeStruct((N, H), jnp.float32)],
    )(x, wcat)


def _mean_from_parts(aggp_ref, degp_ref):
    agg = aggp_ref[0] + aggp_ref[1]
    deg = degp_ref[0][:, 0:1] + degp_ref[1][:, 0:1]
    return agg / jnp.maximum(deg, 1.0)


def _tc_mid_builder(wout, ylw):
    def body(aggp_ref, degp_ref, yr_ref, b_ref, w_ref, yl_ref, yr2_ref):
        mean = _mean_from_parts(aggp_ref, degp_ref)
        h = jnp.maximum(mean + yr_ref[...] + b_ref[...], 0.0)
        y = jnp.dot(h, w_ref[...], preferred_element_type=jnp.float32)
        yl = y[:, :wout]
        if ylw > wout:
            yl = jnp.concatenate(
                [yl, jnp.zeros((yl.shape[0], ylw - wout), jnp.float32)],
                axis=1)
        yl_ref[...] = yl
        yr2_ref[...] = y[:, wout:]

    def run(aggp, degp, yr, b, wcat):
        return pl.pallas_call(
            body,
            grid=(N // BN,),
            in_specs=[pl.BlockSpec((NC, BN, H), lambda i: (0, i, 0)),
                      pl.BlockSpec((NC, BN, DW), lambda i: (0, i, 0)),
                      pl.BlockSpec((BN, H), lambda i: (i, 0)),
                      pl.BlockSpec((1, H), lambda i: (0, 0)),
                      pl.BlockSpec((H, 2 * wout), lambda i: (0, 0))],
            out_specs=[pl.BlockSpec((BN, ylw), lambda i: (i, 0)),
                       pl.BlockSpec((BN, wout), lambda i: (i, 0))],
            out_shape=[jax.ShapeDtypeStruct((N, ylw), jnp.float32),
                       jax.ShapeDtypeStruct((N, wout), jnp.float32)],
        )(aggp, degp, yr, b, wcat)

    return run


_tc_mid_h = _tc_mid_builder(H, H)
_tc_mid_cp = _tc_mid_builder(CP, H)


def _tc_fin_body(aggp_ref, degp_ref, yr_ref, b_ref, o_ref):
    mean = _mean_from_parts(aggp_ref, degp_ref)
    logits = mean[:, :CP] + yr_ref[...] + b_ref[...]
    col = lax.broadcasted_iota(jnp.int32, logits.shape, 1)
    neg = jnp.where(col < C, logits, -3e38)
    m = jnp.max(neg, axis=1, keepdims=True)
    lse = jnp.log(jnp.sum(jnp.exp(neg - m), axis=1, keepdims=True)) + m
    o_ref[...] = logits - lse


def _tc_fin(aggp, degp, yr, b):
    return pl.pallas_call(
        _tc_fin_body,
        grid=(N // BN,),
        in_specs=[pl.BlockSpec((NC, BN, H), lambda i: (0, i, 0)),
                  pl.BlockSpec((NC, BN, DW), lambda i: (0, i, 0)),
                  pl.BlockSpec((BN, CP), lambda i: (i, 0)),
                  pl.BlockSpec((1, CP), lambda i: (0, 0))],
        out_specs=pl.BlockSpec((BN, CP), lambda i: (i, 0)),
        out_shape=jax.ShapeDtypeStruct((N, CP), jnp.float32),
    )(aggp, degp, yr, b)


def kernel(x, adj_t, W1l, W1r, b1, W2l, W2r, b2, W3l, W3r, b3):
    src = adj_t[0].astype(jnp.int32)
    dst = adj_t[1].astype(jnp.int32)
    srcp = jnp.concatenate([src, jnp.zeros((EP - E,), jnp.int32)])
    dstp = jnp.concatenate([dst, jnp.full((EP - E,), N, jnp.int32)])
    eidx = jnp.stack([srcp.reshape(NCHUNK, B), dstp.reshape(NCHUNK, B)],
                     axis=1)

    wc1 = jnp.concatenate([W1l.T, W1r.T], axis=1)
    wc2 = jnp.concatenate([W2l.T, W2r.T], axis=1)
    w3lp = jnp.pad(W3l, ((0, CP - C), (0, 0)))
    w3rp = jnp.pad(W3r, ((0, CP - C), (0, 0)))
    wc3 = jnp.concatenate([w3lp.T, w3rp.T], axis=1)
    b1r = b1.reshape(1, H)
    b2r = b2.reshape(1, H)
    b3r = jnp.pad(b3, (0, CP - C)).reshape(1, CP)

    degp = _sc_deg(eidx)
    yl1, yr1 = _tc_in(x, wc1)
    agg1 = _sc_agg(yl1, eidx)
    yl2, yr2 = _tc_mid_h(agg1, degp, yr1, b1r, wc2)
    agg2 = _sc_agg(yl2, eidx)
    yl3, yr3 = _tc_mid_cp(agg2, degp, yr2, b2r, wc3)
    agg3 = _sc_agg(yl3, eidx)
    out = _tc_fin(agg3, degp, yr3, b3r)
    return out[:, :C]
